# baseline (speedup 1.0000x reference)
import functools

import jax
from jax import lax
from jax.experimental import pallas as pl
from jax.experimental.pallas import tpu as pltpu
from jax.experimental.pallas import tpu_sc as plsc


def kernel(x, edge_index):
    del edge_index
    n, d = x.shape
    total = n * d
    flat = x.reshape(total)

    info = plsc.get_sparse_core_info()
    nc, ns = info.num_cores, info.num_subcores
    nw = nc * ns
    chunk = total // nw

    mesh = plsc.VectorSubcoreMesh(core_axis_name="c", subcore_axis_name="s")

    @functools.partial(
        pl.kernel,
        mesh=mesh,
        out_type=jax.ShapeDtypeStruct((total,), x.dtype),
        scratch_types=[pltpu.VMEM((chunk,), x.dtype)],
    )
    def sc_copy(x_hbm, o_hbm, buf):
        wid = lax.axis_index("s") * nc + lax.axis_index("c")
        base = wid * chunk
        pltpu.sync_copy(x_hbm.at[pl.ds(base, chunk)], buf)
        pltpu.sync_copy(buf, o_hbm.at[pl.ds(base, chunk)])

    return sc_copy(flat).reshape(n, d)


# speedup vs baseline: 5.4618x; 5.4618x over previous
import jax
from jax.experimental import pallas as pl
from jax.experimental.pallas import tpu as pltpu

_BLOCK_ROWS = 5000


def _copy_kernel(x_ref, o_ref):
    o_ref[...] = x_ref[...]


def kernel(x, edge_index):
    del edge_index
    n, d = x.shape
    grid = (pl.cdiv(n, _BLOCK_ROWS),)
    return pl.pallas_call(
        _copy_kernel,
        out_shape=jax.ShapeDtypeStruct(x.shape, x.dtype),
        grid=grid,
        in_specs=[pl.BlockSpec((_BLOCK_ROWS, d), lambda i: (i, 0))],
        out_specs=pl.BlockSpec((_BLOCK_ROWS, d), lambda i: (i, 0)),
        compiler_params=pltpu.CompilerParams(
            dimension_semantics=("arbitrary",),
        ),
    )(x)
